# baseline (speedup 1.0000x reference)
import functools

import jax
import jax.numpy as jnp
from jax import lax
from jax.experimental import pallas as pl
from jax.experimental import pallas as pl
from jax.experimental.pallas import tpu as pltpu
from jax.experimental.pallas import tpu_sc as plsc

N = 10000
E = 160000
IN_DIM = 256
HID = 512
NB = 3

NPAD = 10240
QH = NPAD // 4
RPTQ = QH // 16

KQ = 128
GRP = 8
PADQ = KQ * GRP
MT = E + 4 * PADQ
MTB = MT // KQ

K1 = 128
MP = 163840
NB32 = MP // (32 * K1)
RPT = NPAD // 16

R = 1024
GRID = NPAD // R


def _make_propq(width):
    ncb = width // 256
    nsl = width // 128
    mesh = plsc.VectorSubcoreMesh(core_axis_name="c", subcore_axis_name="s")
    scratch = [
        pltpu.VMEM_SHARED((QH + 8, 2, 128), jnp.float32),
        pltpu.VMEM((GRP, KQ), jnp.int32),
        pltpu.VMEM((GRP, KQ), jnp.int32),
        pltpu.VMEM((KQ, 2, 128), jnp.float32),
        pltpu.VMEM((KQ, 2, 128), jnp.float32),
        pltpu.VMEM((16,), jnp.int32),
        pltpu.SemaphoreType.DMA,
        pltpu.SemaphoreType.DMA,
    ]

    @functools.partial(
        pl.kernel, mesh=mesh,
        out_type=jax.ShapeDtypeStruct((NPAD, nsl, 128), jnp.float32),
        scratch_types=scratch)
    def prop(u, srcr, dstr, meta, out,
             acc, isrc, idst, buf0, buf1, smeta, sem0, sem1):
        core = lax.axis_index("c")
        sub = lax.axis_index("s")
        pltpu.sync_copy(meta, smeta)
        bufs = (buf0, buf1)
        sems = (sem0, sem1)
        for sc in range(2):
            @pl.when(core == sc)
            def _(sc=sc):
                for cb in range(ncb):
                    cs = cb * 2
                    for p in range(2):
                        q = sc * 2 + p
                        mv = smeta[...]
                        nbq = mv[q]
                        offb = mv[4 + q]
                        pltpu.sync_copy(
                            u.at[pl.ds(q * QH + sub * RPTQ, RPTQ),
                                 pl.ds(cs, 2)],
                            acc.at[pl.ds(sub * RPTQ, RPTQ)])
                        plsc.subcore_barrier()
                        lo = offb + (nbq * sub // 16 // GRP) * GRP
                        hi = offb + (nbq * (sub + 1) // 16 // GRP) * GRP
                        ngrp = (hi - lo) // GRP

                        def grp_body(g, carry, cs=cs):
                            row = pl.multiple_of(lo + g * GRP, GRP)
                            pltpu.sync_copy(srcr.at[pl.ds(row, GRP)], isrc)
                            pltpu.sync_copy(dstr.at[pl.ds(row, GRP)], idst)
                            pltpu.async_copy(
                                u.at[isrc.at[0], pl.ds(cs, 2)],
                                bufs[0], sems[0])
                            for i in range(GRP):
                                b = i % 2
                                pltpu.make_async_copy(
                                    u.at[isrc.at[i], pl.ds(cs, 2)],
                                    bufs[b], sems[b]).wait()
                                if i + 1 < GRP:
                                    pltpu.async_copy(
                                        u.at[isrc.at[i + 1], pl.ds(cs, 2)],
                                        bufs[1 - b], sems[1 - b])
                                pltpu.sync_copy(bufs[b],
                                                acc.at[idst.at[i]],
                                                add=True)
                            return carry

                        lax.fori_loop(0, ngrp, grp_body, 0)
                        plsc.subcore_barrier()
                        pltpu.sync_copy(
                            acc.at[pl.ds(sub * RPTQ, RPTQ)],
                            out.at[pl.ds(q * QH + sub * RPTQ, RPTQ),
                                   pl.ds(cs, 2)])
                        plsc.subcore_barrier()

    return prop


_propq256 = _make_propq(256)
_propq512 = _make_propq(512)


def _make_prop_split():
    mesh = plsc.VectorSubcoreMesh(core_axis_name="c", subcore_axis_name="s")
    out_type = [jax.ShapeDtypeStruct((NPAD, 128), jnp.float32)] * 2
    scratch = [
        pltpu.VMEM_SHARED((NPAD + 16, 128), jnp.float32),
        pltpu.VMEM((NB32, K1), jnp.int32),
        pltpu.VMEM((NB32, K1), jnp.int32),
        pltpu.VMEM((K1, 128), jnp.float32),
        pltpu.VMEM((K1, 128), jnp.float32),
        pltpu.SemaphoreType.DMA,
        pltpu.SemaphoreType.DMA,
    ]

    @functools.partial(pl.kernel, out_type=out_type, mesh=mesh,
                       scratch_types=scratch)
    def prop(u, zinit, src_r, dst_r, out_a, out_b,
             acc, isrc, idst, buf0, buf1, sem0, sem1):
        core = lax.axis_index("c")
        sub = lax.axis_index("s")
        w = core * 16 + sub
        bufs = (buf0, buf1)
        sems = (sem0, sem1)
        for sc in range(2):
            @pl.when(core == sc)
            def _(sc=sc):
                init = u if sc == 0 else zinit
                out = (out_a, out_b)[sc]
                pltpu.sync_copy(init.at[pl.ds(sub * RPT, RPT)],
                                acc.at[pl.ds(sub * RPT, RPT)])
                plsc.subcore_barrier()
                pltpu.sync_copy(src_r.at[w], isrc)
                pltpu.sync_copy(dst_r.at[w], idst)
                pltpu.async_copy(u.at[isrc.at[0]], bufs[0], sems[0])

                def body(g, carry):
                    for b in range(2):
                        j = g * 2 + b
                        pltpu.make_async_copy(u.at[isrc.at[j]], bufs[b],
                                              sems[b]).wait()

                        @pl.when(j + 1 < NB32)
                        def _():
                            pltpu.async_copy(u.at[isrc.at[j + 1]],
                                             bufs[1 - b], sems[1 - b])

                        pltpu.sync_copy(bufs[b], acc.at[idst.at[j]], add=True)
                    return carry

                lax.fori_loop(0, NB32 // 2, body, 0)
                plsc.subcore_barrier()
                pltpu.sync_copy(acc.at[pl.ds(sub * RPT, RPT)],
                                out.at[pl.ds(sub * RPT, RPT)])
                plsc.subcore_barrier()

    return prop


_prop1 = _make_prop_split()


def _row_spec(width):
    return pl.BlockSpec((R, width), lambda i: (i, 0))


def _full_spec(a, b):
    return pl.BlockSpec((a, b), lambda i: (0, 0))


def _p0_body(deg_ref, x_ref, dinv_ref, xs_ref):
    dv = lax.rsqrt(deg_ref[...])
    dinv_ref[...] = jnp.broadcast_to(dv, (R, 128))
    xs_ref[...] = x_ref[...] * dv


_p0 = pl.pallas_call(
    _p0_body,
    grid=(GRID,),
    in_specs=[pl.BlockSpec((R, 1), lambda i: (i, 0)), _row_spec(IN_DIM)],
    out_specs=[_row_spec(128), _row_spec(IN_DIM)],
    out_shape=[jax.ShapeDtypeStruct((NPAD, 128), jnp.float32),
               jax.ShapeDtypeStruct((NPAD, IN_DIM), jnp.float32)],
)


def _m1_body(s, dinv, Win, bin_, W1, h_ref, u_ref):
    dv = dinv[...][:, :1]
    g = s[...] * dv
    h = jnp.maximum(
        jnp.dot(g, Win[...], preferred_element_type=jnp.float32) + bin_[...],
        0.0)
    h_ref[...] = h
    u_ref[...] = jnp.dot(h, W1[...], preferred_element_type=jnp.float32) * dv


_m1 = pl.pallas_call(
    _m1_body,
    grid=(GRID,),
    in_specs=[_row_spec(IN_DIM), _row_spec(128), _full_spec(IN_DIM, HID),
              _full_spec(1, HID), _full_spec(HID, HID)],
    out_specs=[_row_spec(HID), _row_spec(HID)],
    out_shape=[jax.ShapeDtypeStruct((NPAD, HID), jnp.float32)] * 2,
)


def _mid_body(s, dinv, b, W, u_ref):
    dv = dinv[...][:, :1]
    o = jnp.maximum(s[...] * dv + b[...], 0.0)
    u_ref[...] = jnp.dot(o, W[...], preferred_element_type=jnp.float32) * dv


_mid = pl.pallas_call(
    _mid_body,
    grid=(GRID,),
    in_specs=[_row_spec(HID), _row_spec(128), _full_spec(1, HID),
              _full_spec(HID, HID)],
    out_specs=_row_spec(HID),
    out_shape=jax.ShapeDtypeStruct((NPAD, HID), jnp.float32),
)


def _res_body(s, dinv, b, hres, W, h_ref, u_ref):
    dv = dinv[...][:, :1]
    h = jnp.maximum(s[...] * dv + b[...] + hres[...], 0.0)
    h_ref[...] = h
    u_ref[...] = jnp.dot(h, W[...], preferred_element_type=jnp.float32) * dv


_res = pl.pallas_call(
    _res_body,
    grid=(GRID,),
    in_specs=[_row_spec(HID), _row_spec(128), _full_spec(1, HID),
              _row_spec(HID), _full_spec(HID, HID)],
    out_specs=[_row_spec(HID), _row_spec(HID)],
    out_shape=[jax.ShapeDtypeStruct((NPAD, HID), jnp.float32)] * 2,
)


def _resout_body(s, dinv, b, hres, W, t_ref):
    dv = dinv[...][:, :1]
    h = jnp.maximum(s[...] * dv + b[...] + hres[...], 0.0)
    t_ref[...] = jnp.dot(h, W[...], preferred_element_type=jnp.float32) * dv


_resout = pl.pallas_call(
    _resout_body,
    grid=(GRID,),
    in_specs=[_row_spec(HID), _row_spec(128), _full_spec(1, HID),
              _row_spec(HID), _full_spec(HID, 128)],
    out_specs=_row_spec(128),
    out_shape=jax.ShapeDtypeStruct((NPAD, 128), jnp.float32),
)


def _m8_body(sa, sb, dinv, b, o):
    o[...] = (sa[...] + sb[...]) * dinv[...][:, :1] + b[...]


_m8 = pl.pallas_call(
    _m8_body,
    grid=(GRID,),
    in_specs=[_row_spec(128)] * 3 + [_full_spec(1, 128)],
    out_specs=_row_spec(128),
    out_shape=jax.ShapeDtypeStruct((NPAD, 128), jnp.float32),
)


def kernel(x, edge_index, W_in, b_in, Wb1, bb1, Wb2, bb2, W_out, b_out):
    src, dst = edge_index[0], edge_index[1]
    dst_s, src_s = lax.sort((dst, src), num_keys=1)
    ideg = jnp.zeros((N,), jnp.int32).at[dst].add(1)
    deg = (ideg + 1).astype(jnp.float32)
    deg_p = jnp.concatenate(
        [deg, jnp.ones((NPAD - N,), jnp.float32)]).reshape(NPAD, 1)

    q_of = dst_s // QH
    cnt = jnp.zeros((4,), jnp.int32).at[q_of].add(1)
    pq = ((cnt + (PADQ - 1)) // PADQ) * PADQ
    offe = jnp.concatenate([jnp.zeros((1,), jnp.int32), jnp.cumsum(pq)])[:4]
    starts = jnp.concatenate(
        [jnp.zeros((1,), jnp.int32), jnp.cumsum(cnt)])[:4]
    eidx = jnp.arange(E, dtype=jnp.int32)
    pos = offe[q_of] + (eidx - starts[q_of])
    srcq = jnp.zeros((MT,), jnp.int32).at[pos].set(src_s).reshape(MTB, KQ)
    dstq = jnp.full((MT,), QH, jnp.int32).at[pos].set(
        dst_s - q_of * QH).reshape(MTB, KQ)
    meta = jnp.concatenate([pq // KQ, offe // KQ,
                            jnp.zeros((8,), jnp.int32)])

    pad_e = MP - E
    src_b = jnp.concatenate(
        [src_s, jnp.zeros((pad_e,), jnp.int32)]).reshape(32, NB32, K1)
    dst_b = jnp.concatenate(
        [dst_s, jnp.full((pad_e,), NPAD, jnp.int32)]).reshape(32, NB32, K1)

    xp = jnp.concatenate([x, jnp.zeros((NPAD - N, IN_DIM), jnp.float32)])
    zeros128 = jnp.zeros((NPAD, 128), jnp.float32)
    W_out_p = jnp.concatenate(
        [W_out, jnp.zeros((HID, 128 - W_out.shape[1]), jnp.float32)], axis=1)
    b_out_p = jnp.concatenate(
        [b_out, jnp.zeros((128 - b_out.shape[0],), jnp.float32)]).reshape(1, 128)
    b_in_r = b_in.reshape(1, HID)

    dinv, xs = _p0(deg_p, xp)
    s = _propq256(xs.reshape(NPAD, 2, 128), srcq, dstq,
                  meta).reshape(NPAD, IN_DIM)
    h, u = _m1(s, dinv, W_in, b_in_r, Wb1[0])
    for i in range(NB):
        s = _propq512(u.reshape(NPAD, 4, 128), srcq, dstq,
                      meta).reshape(NPAD, HID)
        u = _mid(s, dinv, bb1[i].reshape(1, HID), Wb2[i])
        s = _propq512(u.reshape(NPAD, 4, 128), srcq, dstq,
                      meta).reshape(NPAD, HID)
        if i < NB - 1:
            h, u = _res(s, dinv, bb2[i].reshape(1, HID), h, Wb1[i + 1])
        else:
            t = _resout(s, dinv, bb2[i].reshape(1, HID), h, W_out_p)
    sa, sb = _prop1(t, zeros128, src_b, dst_b)
    y = _m8(sa, sb, dinv, b_out_p)
    return y[:N, :W_out.shape[1]]


# speedup vs baseline: 1.0504x; 1.0504x over previous
import functools

import jax
import jax.numpy as jnp
from jax import lax
from jax.experimental import pallas as pl
from jax.experimental.pallas import tpu as pltpu
from jax.experimental.pallas import tpu_sc as plsc

N = 10000
E = 160000
IN_DIM = 256
HID = 512
NB = 3

NPAD = 10240
K = 128
DEPTH = 2
MP = 163840
NB16 = MP // (16 * K)
NB32 = MP // (32 * K)
RPT = NPAD // 16

R = 1024
GRID = NPAD // R


def _prop_body(u, out, acc, isrc, idst, src_rs, dst_rs,
               bufs, sems, ssems, sub, nb, init=None, halves=2):
    if init is None:
        init = u
    gb = nb // halves
    pltpu.sync_copy(init.at[pl.ds(sub * RPT, RPT)],
                    acc.at[pl.ds(sub * RPT, RPT)])
    plsc.subcore_barrier()
    for half in range(halves):
        pltpu.sync_copy(src_rs.at[pl.ds(half * gb, gb)], isrc)
        pltpu.sync_copy(dst_rs.at[pl.ds(half * gb, gb)], idst)
        for d in range(DEPTH - 1):
            pltpu.async_copy(u.at[isrc.at[d]], bufs[d], sems[d])

        def body(g, carry):
            for b in range(DEPTH):
                j = g * DEPTH + b
                pltpu.make_async_copy(u.at[isrc.at[j]], bufs[b],
                                      sems[b]).wait()
                pltpu.async_copy(bufs[b], acc.at[idst.at[j]], ssems[b],
                                 add=True)
                bn = (b + 1) % DEPTH

                @pl.when(jnp.logical_and(j + 1 < gb, j + 1 >= DEPTH))
                def _():
                    pltpu.make_async_copy(bufs[bn], acc.at[idst.at[j]],
                                          ssems[bn]).wait()

                @pl.when(j + 1 < gb)
                def _():
                    pltpu.async_copy(u.at[isrc.at[j + 1]],
                                     bufs[bn], sems[bn])
            return carry

        lax.fori_loop(0, gb // DEPTH, body, 0)
        for d in range(DEPTH):
            pltpu.make_async_copy(bufs[d], acc.at[idst.at[gb - DEPTH + d]],
                                  ssems[d]).wait()
    plsc.subcore_barrier()
    pltpu.sync_copy(acc.at[pl.ds(sub * RPT, RPT)], out.at[pl.ds(sub * RPT, RPT)])
    plsc.subcore_barrier()


def _make_prop(cb):
    passes = cb // 2
    mesh = plsc.VectorSubcoreMesh(core_axis_name="c", subcore_axis_name="s")
    out_type = [jax.ShapeDtypeStruct((NPAD, 128), jnp.float32)] * cb
    scratch = [
        pltpu.VMEM_SHARED((NPAD + 16, 128), jnp.float32),
        pltpu.VMEM((NB16 // 2, K), jnp.int32),
        pltpu.VMEM((NB16 // 2, K), jnp.int32),
    ] + [pltpu.VMEM((K, 128), jnp.float32)] * DEPTH \
      + [pltpu.SemaphoreType.DMA] * (2 * DEPTH)

    @functools.partial(pl.kernel, out_type=out_type, mesh=mesh,
                       scratch_types=scratch)
    def prop(*refs):
        us = refs[:cb]
        src_r, dst_r = refs[cb], refs[cb + 1]
        outs = refs[cb + 2:2 * cb + 2]
        acc, isrc, idst = refs[2 * cb + 2:2 * cb + 5]
        bufs = refs[2 * cb + 5:2 * cb + 5 + DEPTH]
        sems = refs[2 * cb + 5 + DEPTH:2 * cb + 5 + 2 * DEPTH]
        ssems = refs[2 * cb + 5 + 2 * DEPTH:2 * cb + 5 + 3 * DEPTH]
        core = lax.axis_index("c")
        sub = lax.axis_index("s")
        for sc in range(2):
            @pl.when(core == sc)
            def _(sc=sc):
                for p in range(passes):
                    _prop_body(us[sc * passes + p], outs[sc * passes + p],
                               acc, isrc, idst, src_r.at[sub], dst_r.at[sub],
                               bufs, sems, ssems, sub, NB16, halves=2)

    return prop


def _make_prop_split():
    mesh = plsc.VectorSubcoreMesh(core_axis_name="c", subcore_axis_name="s")
    out_type = [jax.ShapeDtypeStruct((NPAD, 128), jnp.float32)] * 2
    scratch = [
        pltpu.VMEM_SHARED((NPAD + 16, 128), jnp.float32),
        pltpu.VMEM((NB32, K), jnp.int32),
        pltpu.VMEM((NB32, K), jnp.int32),
    ] + [pltpu.VMEM((K, 128), jnp.float32)] * DEPTH \
      + [pltpu.SemaphoreType.DMA] * (2 * DEPTH)

    @functools.partial(pl.kernel, out_type=out_type, mesh=mesh,
                       scratch_types=scratch)
    def prop(u, zinit, src_r, dst_r, out_a, out_b,
             acc, isrc, idst, *bufsems):
        bufs = bufsems[:DEPTH]
        sems = bufsems[DEPTH:2 * DEPTH]
        ssems = bufsems[2 * DEPTH:]
        core = lax.axis_index("c")
        sub = lax.axis_index("s")
        w = core * 16 + sub
        for sc in range(2):
            @pl.when(core == sc)
            def _(sc=sc):
                init = u if sc == 0 else zinit
                out = (out_a, out_b)[sc]
                _prop_body(u, out, acc, isrc, idst,
                           src_r.at[w], dst_r.at[w],
                           bufs, sems, ssems, sub, NB32,
                           init=init, halves=1)

    return prop


_prop2 = _make_prop(2)
_prop4 = _make_prop(4)
_prop1 = _make_prop_split()


def _row_spec(width):
    return pl.BlockSpec((R, width), lambda i: (i, 0))


def _full_spec(a, b):
    return pl.BlockSpec((a, b), lambda i: (0, 0))


def _o128(n):
    return [jax.ShapeDtypeStruct((NPAD, 128), jnp.float32)] * n


def _p0_body(deg_ref, x_ref, dinv_ref, o0, o1):
    dv = lax.rsqrt(deg_ref[...])
    dinv_ref[...] = jnp.broadcast_to(dv, (R, 128))
    o0[...] = x_ref[:, :128] * dv
    o1[...] = x_ref[:, 128:] * dv


_p0 = pl.pallas_call(
    _p0_body,
    grid=(GRID,),
    in_specs=[pl.BlockSpec((R, 1), lambda i: (i, 0)), _row_spec(256)],
    out_specs=[_row_spec(128)] * 3,
    out_shape=_o128(3),
)


def _m1_body(s0, s1, dinv, Win, bin_, W1, h_ref, o0, o1, o2, o3):
    dv = dinv[...]
    g = jnp.concatenate([s0[...] * dv, s1[...] * dv], axis=1)
    h = jnp.maximum(
        jnp.dot(g, Win[...], preferred_element_type=jnp.float32) + bin_[...],
        0.0)
    h_ref[...] = h
    u = jnp.dot(h, W1[...], preferred_element_type=jnp.float32)
    for cbi, o in enumerate((o0, o1, o2, o3)):
        o[...] = u[:, cbi * 128:(cbi + 1) * 128] * dv


_m1 = pl.pallas_call(
    _m1_body,
    grid=(GRID,),
    in_specs=[_row_spec(128)] * 3 + [_full_spec(IN_DIM, HID),
                                     _full_spec(1, HID),
                                     _full_spec(HID, HID)],
    out_specs=[_row_spec(HID)] + [_row_spec(128)] * 4,
    out_shape=[jax.ShapeDtypeStruct((NPAD, HID), jnp.float32)] + _o128(4),
)


def _mid_body(s0, s1, s2, s3, dinv, b, W, o0, o1, o2, o3):
    dv = dinv[...]
    g = jnp.concatenate([s[...] * dv for s in (s0, s1, s2, s3)], axis=1)
    o = jnp.maximum(g + b[...], 0.0)
    u = jnp.dot(o, W[...], preferred_element_type=jnp.float32)
    for cbi, oref in enumerate((o0, o1, o2, o3)):
        oref[...] = u[:, cbi * 128:(cbi + 1) * 128] * dv


_mid = pl.pallas_call(
    _mid_body,
    grid=(GRID,),
    in_specs=[_row_spec(128)] * 5 + [_full_spec(1, HID), _full_spec(HID, HID)],
    out_specs=[_row_spec(128)] * 4,
    out_shape=_o128(4),
)


def _res_body(s0, s1, s2, s3, dinv, b, hres, W, h_ref, o0, o1, o2, o3):
    dv = dinv[...]
    g = jnp.concatenate([s[...] * dv for s in (s0, s1, s2, s3)], axis=1)
    h = jnp.maximum(g + b[...] + hres[...], 0.0)
    h_ref[...] = h
    u = jnp.dot(h, W[...], preferred_element_type=jnp.float32)
    for cbi, oref in enumerate((o0, o1, o2, o3)):
        oref[...] = u[:, cbi * 128:(cbi + 1) * 128] * dv


_res = pl.pallas_call(
    _res_body,
    grid=(GRID,),
    in_specs=[_row_spec(128)] * 5 + [_full_spec(1, HID), _row_spec(HID),
                                     _full_spec(HID, HID)],
    out_specs=[_row_spec(HID)] + [_row_spec(128)] * 4,
    out_shape=[jax.ShapeDtypeStruct((NPAD, HID), jnp.float32)] + _o128(4),
)


def _resout_body(s0, s1, s2, s3, dinv, b, hres, W, o0):
    dv = dinv[...]
    g = jnp.concatenate([s[...] * dv for s in (s0, s1, s2, s3)], axis=1)
    h = jnp.maximum(g + b[...] + hres[...], 0.0)
    u = jnp.dot(h, W[...], preferred_element_type=jnp.float32)
    o0[...] = u * dv


_resout = pl.pallas_call(
    _resout_body,
    grid=(GRID,),
    in_specs=[_row_spec(128)] * 5 + [_full_spec(1, HID), _row_spec(HID),
                                     _full_spec(HID, 128)],
    out_specs=_row_spec(128),
    out_shape=_o128(1)[0],
)


def _m8_body(sa, sb, dinv, b, o):
    o[...] = (sa[...] + sb[...]) * dinv[...] + b[...]


_m8 = pl.pallas_call(
    _m8_body,
    grid=(GRID,),
    in_specs=[_row_spec(128)] * 3 + [_full_spec(1, 128)],
    out_specs=_row_spec(128),
    out_shape=_o128(1)[0],
)


def kernel(x, edge_index, W_in, b_in, Wb1, bb1, Wb2, bb2, W_out, b_out):
    src, dst = edge_index[0], edge_index[1]
    dst_s, src_s = lax.sort((dst, src), num_keys=1)
    deg = (jnp.zeros((N,), jnp.int32).at[dst].add(1) + 1).astype(jnp.float32)
    deg_p = jnp.concatenate(
        [deg, jnp.ones((NPAD - N,), jnp.float32)]).reshape(NPAD, 1)

    pad_e = MP - E
    src_p = jnp.concatenate([src_s, jnp.zeros((pad_e,), jnp.int32)])
    dst_p = jnp.concatenate([dst_s, jnp.full((pad_e,), NPAD, jnp.int32)])
    src_a = src_p.reshape(16, NB16, K)
    dst_a = dst_p.reshape(16, NB16, K)
    src_b = src_p.reshape(32, NB32, K)
    dst_b = dst_p.reshape(32, NB32, K)

    xp = jnp.concatenate([x, jnp.zeros((NPAD - N, IN_DIM), jnp.float32)])
    zeros128 = jnp.zeros((NPAD, 128), jnp.float32)
    W_out_p = jnp.concatenate(
        [W_out, jnp.zeros((HID, 128 - W_out.shape[1]), jnp.float32)], axis=1)
    b_out_p = jnp.concatenate(
        [b_out, jnp.zeros((128 - b_out.shape[0],), jnp.float32)]).reshape(1, 128)
    b_in_r = b_in.reshape(1, HID)

    dinv, x0, x1 = _p0(deg_p, xp)
    s0, s1 = _prop2(x0, x1, src_a, dst_a)
    h, *u = _m1(s0, s1, dinv, W_in, b_in_r, Wb1[0])
    for i in range(NB):
        s = _prop4(*u, src_a, dst_a)
        u = _mid(*s, dinv, bb1[i].reshape(1, HID), Wb2[i])
        s = _prop4(*u, src_a, dst_a)
        if i < NB - 1:
            h, *u = _res(*s, dinv, bb2[i].reshape(1, HID), h, Wb1[i + 1])
        else:
            t = _resout(*s, dinv, bb2[i].reshape(1, HID), h, W_out_p)
    sa, sb = _prop1(t, zeros128, src_b, dst_b)
    y = _m8(sa, sb, dinv, b_out_p)
    return y[:N, :W_out.shape[1]]
